# Initial kernel scaffold; baseline (speedup 1.0000x reference)
import jax
import jax.numpy as jnp
from jax.experimental import pallas as pl


def kernel(x, token_table, pe):
    raise NotImplementedError("write your pallas kernel here")


# speedup vs baseline: 1.1715x; 1.1715x over previous
import functools

import jax
import jax.numpy as jnp
from jax import lax
from jax.experimental import pallas as pl
from jax.experimental.pallas import tpu as pltpu
from jax.experimental.pallas import tpu_sc as plsc

LANES = 16
NC = 2
NS = 16
NW = NC * NS


@functools.partial(jax.jit, static_argnums=(3, 4, 5, 6))
def _embed_add(x_flat, token_table, pe, N, S, D, B):
    CP = 32
    pos_per_w = S // NW
    n_pchunks = pos_per_w // CP
    n_steps = n_pchunks * B
    nv = D // LANES
    mesh = plsc.VectorSubcoreMesh(core_axis_name="c", subcore_axis_name="s")

    @functools.partial(
        pl.kernel,
        mesh=mesh,
        out_type=jax.ShapeDtypeStruct((N, D), jnp.float32),
        scratch_types=[
            pltpu.VMEM((B * pos_per_w,), jnp.int32),
            pltpu.VMEM((CP, D), jnp.float32),
            pltpu.VMEM((CP, D), jnp.float32),
            pltpu.VMEM((CP, D), jnp.float32),
            pltpu.VMEM((CP, D), jnp.float32),
            pltpu.SemaphoreType.DMA,
            pltpu.SemaphoreType.DMA,
            pltpu.SemaphoreType.DMA,
            pltpu.SemaphoreType.DMA,
            pltpu.SemaphoreType.DMA,
            pltpu.SemaphoreType.DMA,
        ],
    )
    def k(x_hbm, table_hbm, pe_hbm, out_hbm,
          idx_v, rows0, rows1, pe0, pe1,
          rsem0, rsem1, psem0, psem1, ssem0, ssem1):
        wid = lax.axis_index("s") * NC + lax.axis_index("c")
        wpos = wid * pos_per_w
        rows = (rows0, rows1)
        pes = (pe0, pe1)
        rsems = (rsem0, rsem1)
        psems = (psem0, psem1)
        ssems = (ssem0, ssem1)

        for b in range(B):
            pltpu.sync_copy(x_hbm.at[pl.ds(b * S + wpos, pos_per_w)],
                            idx_v.at[pl.ds(b * pos_per_w, pos_per_w)])

        def gather_start(g):
            p, b = divmod(g, B)
            buf = g % 2
            idx_sl = idx_v.at[pl.ds(b * pos_per_w + p * CP, CP)]
            return pltpu.async_copy(table_hbm.at[idx_sl], rows[buf], rsems[buf])

        def pe_start(p):
            buf = p % 2
            return pltpu.async_copy(pe_hbm.at[pl.ds(wpos + p * CP, CP)],
                                    pes[buf], psems[buf])

        pe_copies = {0: pe_start(0)}
        row_copies = {0: gather_start(0)}
        store_copies = {}

        for g in range(n_steps):
            p, b = divmod(g, B)
            buf = g % 2
            if g + 1 < n_steps:
                p1, b1 = divmod(g + 1, B)
                if b1 == 0:
                    pe_copies[p1] = pe_start(p1)
                if g - 1 >= 0:
                    store_copies[g - 1].wait()
                row_copies[g + 1] = gather_start(g + 1)
            row_copies[g].wait()
            if b == 0:
                pe_copies[p].wait()

            pe_buf = pes[p % 2]
            row_buf = rows[buf]

            def add_row(r, c):
                for v in range(nv):
                    sl = pl.ds(v * LANES, LANES)
                    plsc.addupdate(row_buf.at[r, sl], pe_buf[r, sl])
                return c

            lax.fori_loop(0, CP, add_row, 0)

            base = b * S + wpos + p * CP
            store_copies[g] = pltpu.async_copy(
                row_buf, out_hbm.at[pl.ds(base, CP)], ssems[buf])

        store_copies[n_steps - 2].wait()
        store_copies[n_steps - 1].wait()

    return k(x_flat, token_table, pe)


def kernel(x, token_table, pe):
    B, S = x.shape
    D = token_table.shape[1]
    N = B * S
    x_flat = x.reshape(N).astype(jnp.int32)
    out = _embed_add(x_flat, token_table, pe, N, S, D, B)
    return out.reshape(B, S, D)


# speedup vs baseline: 1.2721x; 1.0859x over previous
import functools

import jax
import jax.numpy as jnp
from jax import lax
from jax.experimental import pallas as pl
from jax.experimental.pallas import tpu as pltpu
from jax.experimental.pallas import tpu_sc as plsc

LANES = 16
NC = 2
NS = 16
NW = NC * NS


@functools.partial(jax.jit, static_argnums=(3, 4, 5, 6))
def _embed_add(x_flat, token_table, pe, N, S, D, B):
    CP = 32
    pos_per_w = S // NW
    n_p = pos_per_w // CP
    n_steps = n_p * B
    nv = D // LANES
    mesh = plsc.VectorSubcoreMesh(core_axis_name="c", subcore_axis_name="s")

    @functools.partial(
        pl.kernel,
        mesh=mesh,
        out_type=jax.ShapeDtypeStruct((N, D), jnp.float32),
        scratch_types=[
            pltpu.VMEM((B * pos_per_w,), jnp.int32),
            pltpu.VMEM((2 * CP, D), jnp.float32),
            pltpu.VMEM((2 * CP, D), jnp.float32),
            pltpu.SemaphoreType.DMA((2,)),
            pltpu.SemaphoreType.DMA((2,)),
            pltpu.SemaphoreType.DMA((2,)),
        ],
    )
    def k(x_hbm, table_hbm, pe_hbm, out_hbm,
          idx_v, rowsb, peb, gsem, psem, ssem):
        wid = lax.axis_index("s") * NC + lax.axis_index("c")
        wpos = wid * pos_per_w

        for b in range(B):
            pltpu.sync_copy(x_hbm.at[pl.ds(b * S + wpos, pos_per_w)],
                            idx_v.at[pl.ds(b * pos_per_w, pos_per_w)])

        def start_gather(g):
            p = g >> 2
            b = g & 3
            buf = g & 1
            idx_sl = idx_v.at[pl.ds(b * pos_per_w + p * CP, CP)]
            pltpu.async_copy(table_hbm.at[idx_sl],
                             rowsb.at[pl.ds(buf * CP, CP)], gsem.at[buf])

        def start_pe(p):
            pb = p & 1
            pltpu.async_copy(pe_hbm.at[pl.ds(wpos + p * CP, CP)],
                             peb.at[pl.ds(pb * CP, CP)], psem.at[pb])

        def drain(sem_entry):
            pltpu.make_async_copy(out_hbm.at[pl.ds(0, CP)],
                                  rowsb.at[pl.ds(0, CP)], sem_entry).wait()

        start_pe(0)
        start_gather(0)

        def body(g, carry):
            p = g >> 2
            b = g & 3
            buf = g & 1
            pb = p & 1

            @pl.when(g >= 1)
            def _():
                drain(ssem.at[(g + 1) & 1])

            @pl.when(g < n_steps - 1)
            def _():
                start_gather(g + 1)

            @pl.when(jnp.logical_and(b == 3, g < n_steps - 1))
            def _():
                start_pe(p + 1)

            drain(gsem.at[buf])

            @pl.when(b == 0)
            def _():
                drain(psem.at[pb])

            rbase = buf * CP
            pbase = pb * CP

            @plsc.parallel_loop(0, CP, unroll=2)
            def add_row(r):
                for v in range(nv):
                    sl = pl.ds(v * LANES, LANES)
                    plsc.addupdate(rowsb.at[rbase + r, sl],
                                   peb[pbase + r, sl])

            base = b * S + wpos + p * CP
            pltpu.async_copy(rowsb.at[pl.ds(rbase, CP)],
                             out_hbm.at[pl.ds(base, CP)], ssem.at[buf])
            return carry

        lax.fori_loop(0, n_steps, body, 0)
        drain(ssem.at[(n_steps - 1) & 1])

    return k(x_flat, token_table, pe)


def kernel(x, token_table, pe):
    B, S = x.shape
    D = token_table.shape[1]
    N = B * S
    x_flat = x.reshape(N).astype(jnp.int32)
    out = _embed_add(x_flat, token_table, pe, N, S, D, B)
    return out.reshape(B, S, D)
